# baseline (speedup 1.0000x reference)
import functools

import jax
import jax.numpy as jnp
from jax import lax
from jax.experimental import pallas as pl
from jax.experimental.pallas import tpu as pltpu
from jax.experimental.pallas import tpu_sc as plsc

_NC = 2
_NS = 16
_NW = _NC * _NS

_K = 8
_NBUF = 4


@functools.cache
def _build(B, V, D):
    b_per_w = B // _NW
    n_chunks = b_per_w // _K
    mesh = plsc.VectorSubcoreMesh(core_axis_name="c", subcore_axis_name="s")

    @functools.partial(
        pl.kernel,
        mesh=mesh,
        out_type=jax.ShapeDtypeStruct((B, D), jnp.float32),
        scratch_types=[
            pltpu.VMEM((n_chunks, _K), jnp.int32),
            pltpu.VMEM((_NBUF, _K, D), jnp.float32),
        ] + [pltpu.SemaphoreType.DMA] * (2 * _NBUF),
    )
    def emb(table_hbm, idx_hbm, out_hbm, idx_v, buf, *sems):
        gsems, psems = sems[:_NBUF], sems[_NBUF:]
        wid = lax.axis_index("s") * _NC + lax.axis_index("c")
        base = wid * b_per_w
        pltpu.sync_copy(idx_hbm.at[wid], idx_v)
        for b in range(_NBUF):
            pltpu.async_copy(table_hbm.at[idx_v.at[b]], buf.at[b], gsems[b])

        def group(g, carry):
            for b in range(_NBUF):
                j = g * _NBUF + b
                pltpu.make_async_copy(
                    table_hbm.at[idx_v.at[j]], buf.at[b], gsems[b]
                ).wait()
                pltpu.async_copy(
                    buf.at[b], out_hbm.at[pl.ds(base + j * _K, _K)], psems[b]
                )
                nj = j + _NBUF

                @pl.when(nj < n_chunks)
                def _():
                    pltpu.make_async_copy(
                        buf.at[b], out_hbm.at[pl.ds(base, _K)], psems[b]
                    ).wait()
                    pltpu.async_copy(
                        table_hbm.at[idx_v.at[nj]], buf.at[b], gsems[b]
                    )
            return carry

        lax.fori_loop(0, n_chunks // _NBUF, group, 0)
        for b in range(_NBUF):
            pltpu.make_async_copy(
                buf.at[b], out_hbm.at[pl.ds(base, _K)], psems[b]
            ).wait()

    return emb


def kernel(position_ids, table):
    nb, ns = position_ids.shape
    V, D = table.shape
    B = nb * ns
    idx = position_ids.reshape(_NW, (B // _NW) // _K, _K).astype(jnp.int32)
    out = _build(B, V, D)(table, idx)
    return out.reshape(nb, ns, D)


# speedup vs baseline: 1.0104x; 1.0104x over previous
import functools

import jax
import jax.numpy as jnp
from jax import lax
from jax.experimental import pallas as pl
from jax.experimental.pallas import tpu as pltpu
from jax.experimental.pallas import tpu_sc as plsc

_NC = 2
_NS = 16
_NW = _NC * _NS

_K = 8
_NBUF = 4


@functools.cache
def _build(NB, NS_SEQ, V, D):
    B = NB * NS_SEQ
    b_per_w = B // _NW
    tiles_per_batch = _NW // NB
    n_chunks = b_per_w // _K
    mesh = plsc.VectorSubcoreMesh(core_axis_name="c", subcore_axis_name="s")

    @functools.partial(
        pl.kernel,
        mesh=mesh,
        out_type=jax.ShapeDtypeStruct((B, D), jnp.float32),
        scratch_types=[
            pltpu.VMEM((b_per_w,), jnp.int32),
            pltpu.VMEM((_NBUF, _K, D), jnp.float32),
        ] + [pltpu.SemaphoreType.DMA] * _NBUF,
    )
    def emb(table_hbm, idx_hbm, out_hbm, idx_v, buf, *gsems):
        wid = lax.axis_index("s") * _NC + lax.axis_index("c")
        base = wid * b_per_w
        batch = wid // tiles_per_batch
        col0 = (wid % tiles_per_batch) * b_per_w
        pltpu.sync_copy(idx_hbm.at[batch, pl.ds(col0, b_per_w)], idx_v)
        for b in range(_NBUF):
            pltpu.async_copy(
                table_hbm.at[idx_v.at[pl.ds(b * _K, _K)]], buf.at[b], gsems[b]
            )

        def group(g, carry):
            for b in range(_NBUF):
                j = g * _NBUF + b
                pltpu.make_async_copy(
                    table_hbm.at[idx_v.at[pl.ds(j * _K, _K)]], buf.at[b], gsems[b]
                ).wait()
                pltpu.sync_copy(buf.at[b], out_hbm.at[pl.ds(base + j * _K, _K)])
                nj = j + _NBUF

                @pl.when(nj < n_chunks)
                def _():
                    pltpu.async_copy(
                        table_hbm.at[idx_v.at[pl.ds(nj * _K, _K)]],
                        buf.at[b],
                        gsems[b],
                    )
            return carry

        lax.fori_loop(0, n_chunks // _NBUF, group, 0)

    return emb


def kernel(position_ids, table):
    nb, ns = position_ids.shape
    V, D = table.shape
    idx = position_ids.astype(jnp.int32)
    out = _build(nb, ns, V, D)(table, idx)
    return out.reshape(nb, ns, D)
